# Initial kernel scaffold; baseline (speedup 1.0000x reference)
import jax
import jax.numpy as jnp
from jax.experimental import pallas as pl


def kernel(input, adj, W, b):
    raise NotImplementedError("write your pallas kernel here")


# speedup vs baseline: 1.0180x; 1.0180x over previous
import functools

import jax
import jax.numpy as jnp
from jax.experimental import pallas as pl
from jax.experimental.pallas import tpu as pltpu

_BM = 400


def _gcn_body(x_ref, w_ref, adj_ref, b_ref, o_ref, h_ref):
    @pl.when(pl.program_id(0) == 0)
    def _():
        xw = jnp.dot(
            x_ref[...].astype(jnp.bfloat16),
            w_ref[...].astype(jnp.bfloat16),
            preferred_element_type=jnp.float32,
        )
        h_ref[...] = xw.astype(jnp.bfloat16)

    a = adj_ref[...].astype(jnp.bfloat16)
    o_ref[...] = (
        jnp.dot(a, h_ref[...], preferred_element_type=jnp.float32) + b_ref[...]
    )


def kernel(input, adj, W, b):
    n, in_dim = input.shape
    out_dim = W.shape[1]
    bm = _BM if n % _BM == 0 else n
    grid = (n // bm,)
    b2 = b.reshape(1, out_dim)
    out = pl.pallas_call(
        _gcn_body,
        grid=grid,
        in_specs=[
            pl.BlockSpec((n, in_dim), lambda i: (0, 0)),
            pl.BlockSpec((in_dim, out_dim), lambda i: (0, 0)),
            pl.BlockSpec((bm, n), lambda i: (i, 0)),
            pl.BlockSpec((1, out_dim), lambda i: (0, 0)),
        ],
        out_specs=pl.BlockSpec((bm, out_dim), lambda i: (i, 0)),
        out_shape=jax.ShapeDtypeStruct((n, out_dim), jnp.float32),
        scratch_shapes=[pltpu.VMEM((n, out_dim), jnp.bfloat16)],
    )(input, W, adj, b2)
    return out
